# Initial kernel scaffold; baseline (speedup 1.0000x reference)
import jax
import jax.numpy as jnp
from jax.experimental import pallas as pl


def kernel(pred_h_verts, pred_o_verts, gt_h_verts, gt_o_verts):
    raise NotImplementedError("write your pallas kernel here")


# speedup vs baseline: 10.0174x; 10.0174x over previous
import functools

import jax
import jax.numpy as jnp
from jax import lax
from jax.experimental import pallas as pl
from jax.experimental.pallas import tpu as pltpu
from jax.experimental.pallas import tpu_sc as plsc

BT = 8
VH_RAW = 6890
VO = 2048
TH = 128
NT = 54
VH = TH * NT
THR = 0.2
EPS = 1e-8
BIGC = 1e9

_NSLICE = 4
_HS = VH // _NSLICE
_OS = VO // _NSLICE


def _dist_body(a_ref, bt_ref, rmin_ref, rarg_ref, cmin_ref, carg_ref):
    i = pl.program_id(1)
    a = a_ref[0]
    bt = bt_ref[0]
    d2 = lax.dot_general(a, bt, (((1,), (0,)), ((), ())),
                         preferred_element_type=jnp.float32)
    rmin = jnp.min(d2, axis=1)
    col_iota = lax.broadcasted_iota(jnp.int32, (TH, VO), 1)
    rarg = jnp.min(jnp.where(d2 == rmin[:, None], col_iota, VO), axis=1)
    rmin_ref[0, 0, :] = rmin
    rarg_ref[0, 0, :] = rarg
    cmin_t = jnp.min(d2, axis=0)
    row_iota = lax.broadcasted_iota(jnp.int32, (TH, VO), 0) + i * TH
    carg_t = jnp.min(jnp.where(d2 == cmin_t[None, :], row_iota, VH), axis=0)

    @pl.when(i == 0)
    def _init():
        cmin_ref[0, 0, :] = cmin_t
        carg_ref[0, 0, :] = carg_t

    @pl.when(i != 0)
    def _acc():
        prev = cmin_ref[0, 0, :]
        parg = carg_ref[0, 0, :]
        better = cmin_t < prev
        cmin_ref[0, 0, :] = jnp.where(better, cmin_t, prev)
        carg_ref[0, 0, :] = jnp.where(better, carg_t, parg)


def _loss_body(d2_ref, diff_ref, out_ref):
    d2 = d2_ref[...]
    diff = diff_ref[...]
    basis = jnp.sqrt(jnp.maximum(d2, 1e-12))
    mask = basis < THR
    sel = jnp.sum(mask.astype(jnp.int32), axis=1)
    k = jnp.maximum(
        1, jnp.round(jnp.float32(0.2) * sel.astype(jnp.float32)).astype(jnp.int32))
    bits = lax.bitcast_convert_type(basis, jnp.int32)

    def step(_, lohi):
        lo, hi = lohi
        mid = lo + lax.div(hi - lo, 2)
        cnt = jnp.sum(jnp.where(mask & (bits <= mid[:, None]), 1, 0), axis=1)
        ge = cnt >= k
        return jnp.where(ge, lo, mid + 1), jnp.where(ge, mid, hi)

    lo = jnp.zeros((16,), jnp.int32)
    hi = jnp.full((16,), 0x7F800000, jnp.int32)
    lo, hi = lax.fori_loop(0, 31, step, (lo, hi))
    t = lax.bitcast_convert_type(lo, jnp.float32)
    t = jnp.where(sel > 0, t, jnp.float32(1.0))
    w = jnp.maximum((t[:, None] - basis) / (t[:, None] + EPS), 0.0)
    w2 = w * w
    w4 = jnp.where(mask, w2 * w2, 0.0)
    nrow = jnp.sum(w4 * diff, axis=1)
    drow = jnp.sum(w4, axis=1)
    is_h = lax.iota(jnp.int32, 16) < 8
    l_h = (jnp.sum(jnp.where(is_h, nrow, 0.0))
           / (jnp.sum(jnp.where(is_h, drow, 0.0)) + EPS))
    l_o = (jnp.sum(jnp.where(is_h, 0.0, nrow))
           / (jnp.sum(jnp.where(is_h, 0.0, drow)) + EPS))
    out_ref[...] = jnp.broadcast_to(l_h + l_o, (1, 1))


@functools.lru_cache(maxsize=1)
def _build_gather():
    mesh = plsc.VectorSubcoreMesh(core_axis_name="c", subcore_axis_name="s")
    return functools.partial(
        pl.kernel,
        mesh=mesh,
        out_type=[jax.ShapeDtypeStruct((BT * VH,), jnp.float32),
                  jax.ShapeDtypeStruct((BT * VO,), jnp.float32)],
        scratch_types=_GATHER_SCRATCH,
        compiler_params=pltpu.CompilerParams(needs_layout_passes=False),
    )(_gather_body)


_GATHER_SCRATCH = [
        pltpu.VMEM((VH,), jnp.float32), pltpu.VMEM((VH,), jnp.float32),
        pltpu.VMEM((VH,), jnp.float32), pltpu.VMEM((VH,), jnp.float32),
        pltpu.VMEM((VH,), jnp.float32), pltpu.VMEM((VH,), jnp.float32),
        pltpu.VMEM((VO,), jnp.float32), pltpu.VMEM((VO,), jnp.float32),
        pltpu.VMEM((VO,), jnp.float32), pltpu.VMEM((VO,), jnp.float32),
        pltpu.VMEM((VO,), jnp.float32), pltpu.VMEM((VO,), jnp.float32),
        pltpu.VMEM((_HS,), jnp.int32), pltpu.VMEM((_OS,), jnp.int32),
        pltpu.VMEM((_HS,), jnp.float32), pltpu.VMEM((_OS,), jnp.float32),
]


def _gather_body(ghx, ghy, ghz, phx, phy, phz,
                 gox, goy, goz, pox, poy, poz,
                 idxo, idxh, diffh_out, diffo_out,
                 t_ghx, t_ghy, t_ghz, t_phx, t_phy, t_phz,
                 t_gox, t_goy, t_goz, t_pox, t_poy, t_poz,
                 t_idxo, t_idxh, t_dh, t_do):
    wid = lax.axis_index("s") * 2 + lax.axis_index("c")
    b = wid // _NSLICE
    s = wid % _NSLICE
    hbase = b * VH
    obase = b * VO
    pltpu.sync_copy(ghx.at[pl.ds(hbase, VH)], t_ghx)
    pltpu.sync_copy(ghy.at[pl.ds(hbase, VH)], t_ghy)
    pltpu.sync_copy(ghz.at[pl.ds(hbase, VH)], t_ghz)
    pltpu.sync_copy(phx.at[pl.ds(hbase, VH)], t_phx)
    pltpu.sync_copy(phy.at[pl.ds(hbase, VH)], t_phy)
    pltpu.sync_copy(phz.at[pl.ds(hbase, VH)], t_phz)
    pltpu.sync_copy(gox.at[pl.ds(obase, VO)], t_gox)
    pltpu.sync_copy(goy.at[pl.ds(obase, VO)], t_goy)
    pltpu.sync_copy(goz.at[pl.ds(obase, VO)], t_goz)
    pltpu.sync_copy(pox.at[pl.ds(obase, VO)], t_pox)
    pltpu.sync_copy(poy.at[pl.ds(obase, VO)], t_poy)
    pltpu.sync_copy(poz.at[pl.ds(obase, VO)], t_poz)
    hoff = s * _HS
    ooff = s * _OS
    pltpu.sync_copy(idxo.at[pl.ds(hbase + hoff, _HS)], t_idxo)
    pltpu.sync_copy(idxh.at[pl.ds(obase + ooff, _OS)], t_idxh)

    def hstep(c, carry):
        base = c * 16
        idx = t_idxo[pl.ds(base, 16)]
        gx = plsc.load_gather(t_gox, [idx])
        gy = plsc.load_gather(t_goy, [idx])
        gz = plsc.load_gather(t_goz, [idx])
        px = plsc.load_gather(t_pox, [idx])
        py = plsc.load_gather(t_poy, [idx])
        pz = plsc.load_gather(t_poz, [idx])
        shx = t_ghx[pl.ds(hoff + base, 16)]
        shy = t_ghy[pl.ds(hoff + base, 16)]
        shz = t_ghz[pl.ds(hoff + base, 16)]
        spx = t_phx[pl.ds(hoff + base, 16)]
        spy = t_phy[pl.ds(hoff + base, 16)]
        spz = t_phz[pl.ds(hoff + base, 16)]
        dx = (px - spx) - (gx - shx)
        dy = (py - spy) - (gy - shy)
        dz = (pz - spz) - (gz - shz)
        t_dh[pl.ds(base, 16)] = jnp.abs(dx) + jnp.abs(dy) + jnp.abs(dz)
        return carry

    lax.fori_loop(0, _HS // 16, hstep, 0)

    def ostep(c, carry):
        base = c * 16
        idx = t_idxh[pl.ds(base, 16)]
        gx = plsc.load_gather(t_ghx, [idx])
        gy = plsc.load_gather(t_ghy, [idx])
        gz = plsc.load_gather(t_ghz, [idx])
        px = plsc.load_gather(t_phx, [idx])
        py = plsc.load_gather(t_phy, [idx])
        pz = plsc.load_gather(t_phz, [idx])
        sgx = t_gox[pl.ds(ooff + base, 16)]
        sgy = t_goy[pl.ds(ooff + base, 16)]
        sgz = t_goz[pl.ds(ooff + base, 16)]
        spx = t_pox[pl.ds(ooff + base, 16)]
        spy = t_poy[pl.ds(ooff + base, 16)]
        spz = t_poz[pl.ds(ooff + base, 16)]
        dx = (px - spx) - (gx - sgx)
        dy = (py - spy) - (gy - sgy)
        dz = (pz - spz) - (gz - sgz)
        t_do[pl.ds(base, 16)] = jnp.abs(dx) + jnp.abs(dy) + jnp.abs(dz)
        return carry

    lax.fori_loop(0, _OS // 16, ostep, 0)

    pltpu.sync_copy(t_dh, diffh_out.at[pl.ds(hbase + hoff, _HS)])
    pltpu.sync_copy(t_do, diffo_out.at[pl.ds(obase + ooff, _OS)])


def _dist_call(A, Bt):
    return pl.pallas_call(
        _dist_body,
        grid=(BT, NT),
        in_specs=[
            pl.BlockSpec((1, TH, 8), lambda b, i: (b, i, 0)),
            pl.BlockSpec((1, 8, VO), lambda b, i: (b, 0, 0)),
        ],
        out_specs=[
            pl.BlockSpec((1, 1, TH), lambda b, i: (b * NT + i, 0, 0)),
            pl.BlockSpec((1, 1, TH), lambda b, i: (b * NT + i, 0, 0)),
            pl.BlockSpec((1, 1, VO), lambda b, i: (b, 0, 0)),
            pl.BlockSpec((1, 1, VO), lambda b, i: (b, 0, 0)),
        ],
        out_shape=[
            jax.ShapeDtypeStruct((BT * NT, 1, TH), jnp.float32),
            jax.ShapeDtypeStruct((BT * NT, 1, TH), jnp.int32),
            jax.ShapeDtypeStruct((BT, 1, VO), jnp.float32),
            jax.ShapeDtypeStruct((BT, 1, VO), jnp.int32),
        ],
        compiler_params=pltpu.CompilerParams(
            dimension_semantics=("arbitrary", "arbitrary")),
    )(A, Bt)


def _loss_call(d2all, diffall):
    return pl.pallas_call(
        _loss_body,
        out_shape=jax.ShapeDtypeStruct((1, 1), jnp.float32),
    )(d2all, diffall)


def _run_gather(gh, ph, go, po, idxo, idxh):
    return _build_gather()(
        gh[..., 0].reshape(-1), gh[..., 1].reshape(-1), gh[..., 2].reshape(-1),
        ph[..., 0].reshape(-1), ph[..., 1].reshape(-1), ph[..., 2].reshape(-1),
        go[..., 0].reshape(-1), go[..., 1].reshape(-1), go[..., 2].reshape(-1),
        po[..., 0].reshape(-1), po[..., 1].reshape(-1), po[..., 2].reshape(-1),
        idxo.reshape(-1), idxh.reshape(-1))


def kernel(pred_h_verts, pred_o_verts, gt_h_verts, gt_o_verts):
    pad = VH - VH_RAW
    gh = jnp.pad(gt_h_verts, ((0, 0), (0, pad), (0, 0)), constant_values=BIGC)
    ph = jnp.pad(pred_h_verts, ((0, 0), (0, pad), (0, 0)))
    go = gt_o_verts
    po = pred_o_verts
    a2 = jnp.sum(gh * gh, axis=-1, keepdims=True)
    b2 = jnp.sum(go * go, axis=-1, keepdims=True)
    A = jnp.concatenate([gh, a2, jnp.ones_like(a2), jnp.zeros_like(gh)], axis=-1)
    Bm = jnp.concatenate(
        [-2.0 * go, jnp.ones_like(b2), b2, jnp.zeros_like(go)], axis=-1)
    Bt = jnp.transpose(Bm, (0, 2, 1))

    rmin3, rarg3, cmin3, carg3 = _dist_call(A, Bt)
    rmin2 = rmin3.reshape(BT, VH)
    idxo = rarg3.reshape(BT, VH)
    cmin2 = cmin3.reshape(BT, VO)
    idxh = carg3.reshape(BT, VO)

    diffh_flat, diffo_flat = _run_gather(gh, ph, go, po, idxo, idxh)
    diffh = diffh_flat.reshape(BT, VH)
    diffo = diffo_flat.reshape(BT, VO)

    d2all = jnp.concatenate(
        [rmin2, jnp.pad(cmin2, ((0, 0), (0, VH - VO)), constant_values=1e20)],
        axis=0)
    diffall = jnp.concatenate(
        [diffh, jnp.pad(diffo, ((0, 0), (0, VH - VO)))], axis=0)

    out = _loss_call(d2all, diffall)
    return out[0, 0]
